# baseline (speedup 1.0000x reference)
import functools

import jax
import jax.numpy as jnp
from jax import lax
from jax.experimental import pallas as pl
from jax.experimental.pallas import tpu as pltpu
from jax.experimental.pallas import tpu_sc as plsc

VOCAB = 100000
D = 64
B = 4096
NEG = 5
LANES = 16
NCHUNK = D // LANES

NC = 2
NS = 16
NW = NC * NS
BW = B // NW

_mesh = plsc.VectorSubcoreMesh(core_axis_name="c", subcore_axis_name="s")


@functools.partial(
    pl.kernel,
    out_type=jax.ShapeDtypeStruct((NW, 2, LANES), jnp.float32),
    mesh=_mesh,
    compiler_params=pltpu.CompilerParams(use_tc_tiling_on_sc=False),
    scratch_types=[
        pltpu.VMEM((BW,), jnp.int32),
        pltpu.VMEM((BW,), jnp.int32),
        pltpu.VMEM((BW,), jnp.int32),
        pltpu.VMEM((BW,), jnp.int32),
        pltpu.VMEM((NEG, BW), jnp.int32),
        pltpu.VMEM((NEG, BW), jnp.int32),
        pltpu.VMEM((BW, D), jnp.float32),
        pltpu.VMEM((BW, D), jnp.float32),
        pltpu.VMEM((BW, D), jnp.float32),
        pltpu.VMEM((BW, D), jnp.float32),
        pltpu.VMEM((NEG, BW, D), jnp.float32),
        pltpu.VMEM((NEG, BW, D), jnp.float32),
        pltpu.VMEM((2, LANES), jnp.float32),
        pltpu.SemaphoreType.DMA,
    ],
)
def _nerank_sc(u0_h, u1_h, v0_h, v1_h, n0_h, n1_h,
               ru_h, rv_h, au_h, av_h, out_h,
               iu0, iu1, iv0, iv1, in0, in1,
               ru_v, au_v, rv_v, av_v, rvn_v, avn_v, accb, sem):
    wid = lax.axis_index("s") * NC + lax.axis_index("c")
    base = wid * BW

    pltpu.sync_copy(u0_h.at[pl.ds(base, BW)], iu0)
    pltpu.sync_copy(u1_h.at[pl.ds(base, BW)], iu1)
    pltpu.sync_copy(v0_h.at[pl.ds(base, BW)], iv0)
    pltpu.sync_copy(v1_h.at[pl.ds(base, BW)], iv1)
    pltpu.sync_copy(n0_h.at[:, pl.ds(base, BW)], in0)
    pltpu.sync_copy(n1_h.at[:, pl.ds(base, BW)], in1)

    copies = [
        pltpu.async_copy(ru_h.at[iu0], ru_v, sem),
        pltpu.async_copy(au_h.at[iu1], au_v, sem),
        pltpu.async_copy(rv_h.at[iv0], rv_v, sem),
        pltpu.async_copy(av_h.at[iv1], av_v, sem),
    ]
    for n in range(NEG):
        copies.append(pltpu.async_copy(rv_h.at[in0.at[n]], rvn_v.at[n], sem))
        copies.append(pltpu.async_copy(av_h.at[in1.at[n]], avn_v.at[n], sem))
    for cp in copies:
        cp.wait()

    zeros = jnp.zeros((LANES,), jnp.float32)

    def jbody(j, carry):
        accs = carry[:NCHUNK]
        accn = carry[NCHUNK:]
        outs, outn = [], []
        for c in range(NCHUNK):
            sl = pl.ds(c * LANES, LANES)
            u = ru_v[j, sl] + au_v[j, sl]
            v = rv_v[j, sl] + av_v[j, sl]
            ns = rvn_v[0, j, sl] + avn_v[0, j, sl]
            for n in range(1, NEG):
                ns = ns + rvn_v[n, j, sl] + avn_v[n, j, sl]
            outs.append(accs[c] + u * v)
            outn.append(accn[c] + u * ns)
        return tuple(outs + outn)

    carry = lax.fori_loop(0, BW, jbody, (zeros,) * (2 * NCHUNK))
    svec = carry[0]
    nvec = carry[NCHUNK]
    for c in range(1, NCHUNK):
        svec = svec + carry[c]
        nvec = nvec + carry[NCHUNK + c]
    accb[0, :] = svec
    accb[1, :] = nvec
    pltpu.sync_copy(accb, out_h.at[wid])


def kernel(upos, vpos, npos, batch_size, ru_w, rv_w, au_w, av_w):
    u0 = upos[0].astype(jnp.int32)
    u1 = upos[1].astype(jnp.int32)
    v0 = vpos[0].astype(jnp.int32)
    v1 = vpos[1].astype(jnp.int32)
    n0 = npos[0].astype(jnp.int32).T
    n1 = npos[1].astype(jnp.int32).T
    parts = _nerank_sc(u0, u1, v0, v1, n0, n1, ru_w, rv_w, au_w, av_w)
    score = jnp.sum(parts[:, 0, :])
    neg_score = jnp.sum(parts[:, 1, :])
    return jax.nn.log_sigmoid(score) + jax.nn.log_sigmoid(-neg_score)
